# baseline (speedup 1.0000x reference)
import functools

import jax
import jax.numpy as jnp
from jax.experimental import pallas as pl
from jax.experimental.pallas import tpu as pltpu

_RATIO = 0.05
_R = 256


def _ohem_body(x_ref, t_ref, out_ref, loss_ref, *, nblocks, k):
    i = pl.program_id(0)
    r, c = x_ref.shape
    nq = c // 128
    tgt = t_ref[0, 0, :]
    lane = jax.lax.broadcasted_iota(jnp.int32, (r, 128), 1)
    lmask = lane == (tgt[:, None] & 127)
    tq = tgt[:, None] >> 7

    m128 = x_ref[:, 0:128]
    tsel = x_ref[:, 0:128]
    for q in range(1, nq):
        xc = x_ref[:, 128 * q : 128 * (q + 1)]
        m128 = jnp.maximum(m128, xc)
        tsel = jnp.where(tq == q, xc, tsel)
    m = jnp.max(m128, axis=1, keepdims=True)
    t_logit = jnp.sum(jnp.where(lmask, tsel, 0.0), axis=1)

    e128 = jnp.exp(x_ref[:, 0:128] - m)
    for q in range(1, nq):
        e128 = e128 + jnp.exp(x_ref[:, 128 * q : 128 * (q + 1)] - m)
    s = jnp.sum(e128, axis=1)
    lse = m[:, 0] + jnp.log(s)
    loss = lse - t_logit
    loss_ref[pl.ds(i, 1), :] = loss.reshape(1, -1)

    @pl.when(i == nblocks - 1)
    def _select():
        vals = loss_ref[...]
        bits = jax.lax.bitcast_convert_type(vals, jnp.int32)

        def body(j, lo):
            cand = lo + (1 << (30 - j))
            cnt = jnp.sum((bits >= cand).astype(jnp.int32))
            return jnp.where(cnt >= k, cand, lo)

        thr = jax.lax.fori_loop(0, 31, body, jnp.int32(0))
        tval = jax.lax.bitcast_convert_type(thr, jnp.float32)
        gt = bits > thr
        cnt_gt = jnp.sum(gt.astype(jnp.int32))
        sum_gt = jnp.sum(jnp.where(gt, vals, 0.0))
        out_ref[0, 0] = (sum_gt + (k - cnt_gt).astype(jnp.float32) * tval) / k


@functools.partial(jax.jit, static_argnames=("interpret",))
def kernel(input, target, interpret=False):
    n, c = input.shape
    nblocks = n // _R
    k = max(1, int(n * _RATIO))
    out = pl.pallas_call(
        functools.partial(_ohem_body, nblocks=nblocks, k=k),
        grid=(nblocks,),
        in_specs=[
            pl.BlockSpec((_R, c), lambda i: (i, 0)),
            pl.BlockSpec((1, 1, _R), lambda i: (i, 0, 0)),
        ],
        out_specs=pl.BlockSpec(memory_space=pltpu.SMEM),
        out_shape=jax.ShapeDtypeStruct((1, 1), jnp.float32),
        scratch_shapes=[pltpu.VMEM((nblocks, _R), jnp.float32)],
        compiler_params=pltpu.CompilerParams(
            dimension_semantics=("arbitrary",),
        ),
        interpret=interpret,
    )(input, target.reshape(nblocks, 1, _R))
    return out[0, 0]


# speedup vs baseline: 1.1911x; 1.1911x over previous
import functools

import jax
import jax.numpy as jnp
from jax.experimental import pallas as pl
from jax.experimental.pallas import tpu as pltpu

_RATIO = 0.05
_R = 512


def _ohem_body(x_ref, t_ref, out_ref, loss_ref, *, nblocks, k):
    i = pl.program_id(0)
    r, c = x_ref.shape
    nq = c // 128
    tgt = t_ref[0, 0, :]
    lane = jax.lax.broadcasted_iota(jnp.int32, (r, 128), 1)
    lmask = lane == (tgt[:, None] & 127)
    tq = tgt[:, None] >> 7

    m128 = x_ref[:, 0:128]
    tsel = x_ref[:, 0:128]
    for q in range(1, nq):
        xc = x_ref[:, 128 * q : 128 * (q + 1)]
        m128 = jnp.maximum(m128, xc)
        tsel = jnp.where(tq == q, xc, tsel)
    m = jnp.max(m128, axis=1, keepdims=True)
    t_logit = jnp.sum(jnp.where(lmask, tsel, 0.0), axis=1)

    e128 = jnp.exp(x_ref[:, 0:128] - m)
    for q in range(1, nq):
        e128 = e128 + jnp.exp(x_ref[:, 128 * q : 128 * (q + 1)] - m)
    s = jnp.sum(e128, axis=1)
    lse = m[:, 0] + jnp.log(s)
    loss = lse - t_logit
    loss_ref[pl.ds(i, 1), :] = loss.reshape(1, -1)

    @pl.when(i == nblocks - 1)
    def _select():
        vals = loss_ref[...]
        bits = jax.lax.bitcast_convert_type(vals, jnp.int32)

        def body(j, lo):
            cand = lo + (1 << (30 - j))
            cnt = jnp.sum((bits >= cand).astype(jnp.int32))
            return jnp.where(cnt >= k, cand, lo)

        thr = jax.lax.fori_loop(0, 31, body, jnp.int32(0))
        tval = jax.lax.bitcast_convert_type(thr, jnp.float32)
        gt = bits > thr
        cnt_gt = jnp.sum(gt.astype(jnp.int32))
        sum_gt = jnp.sum(jnp.where(gt, vals, 0.0))
        out_ref[0, 0] = (sum_gt + (k - cnt_gt).astype(jnp.float32) * tval) / k


@functools.partial(jax.jit, static_argnames=("interpret",))
def kernel(input, target, interpret=False):
    n, c = input.shape
    nblocks = n // _R
    k = max(1, int(n * _RATIO))
    out = pl.pallas_call(
        functools.partial(_ohem_body, nblocks=nblocks, k=k),
        grid=(nblocks,),
        in_specs=[
            pl.BlockSpec((_R, c), lambda i: (i, 0)),
            pl.BlockSpec((1, 1, _R), lambda i: (i, 0, 0)),
        ],
        out_specs=pl.BlockSpec(memory_space=pltpu.SMEM),
        out_shape=jax.ShapeDtypeStruct((1, 1), jnp.float32),
        scratch_shapes=[pltpu.VMEM((nblocks, _R), jnp.float32)],
        compiler_params=pltpu.CompilerParams(
            dimension_semantics=("arbitrary",),
        ),
        interpret=interpret,
    )(input, target.reshape(nblocks, 1, _R))
    return out[0, 0]


# speedup vs baseline: 1.3013x; 1.0925x over previous
import functools

import jax
import jax.numpy as jnp
from jax.experimental import pallas as pl
from jax.experimental.pallas import tpu as pltpu

_RATIO = 0.05
_R = 1024


def _ohem_body(x_ref, t_ref, out_ref, loss_ref, *, nblocks, k):
    i = pl.program_id(0)
    r, c = x_ref.shape
    nq = c // 128
    tgt = t_ref[0, 0, :]
    lane = jax.lax.broadcasted_iota(jnp.int32, (r, 128), 1)
    lmask = lane == (tgt[:, None] & 127)
    tq = tgt[:, None] >> 7

    m128 = x_ref[:, 0:128]
    tsel = x_ref[:, 0:128]
    for q in range(1, nq):
        xc = x_ref[:, 128 * q : 128 * (q + 1)]
        m128 = jnp.maximum(m128, xc)
        tsel = jnp.where(tq == q, xc, tsel)
    m = jnp.max(m128, axis=1, keepdims=True)
    t_logit = jnp.sum(jnp.where(lmask, tsel, 0.0), axis=1)

    e128 = jnp.exp(x_ref[:, 0:128] - m)
    for q in range(1, nq):
        e128 = e128 + jnp.exp(x_ref[:, 128 * q : 128 * (q + 1)] - m)
    s = jnp.sum(e128, axis=1)
    lse = m[:, 0] + jnp.log(s)
    loss = lse - t_logit
    loss_ref[pl.ds(i, 1), :] = loss.reshape(1, -1)

    @pl.when(i == nblocks - 1)
    def _select():
        vals = loss_ref[...]
        bits = jax.lax.bitcast_convert_type(vals, jnp.int32)

        def body(j, lo):
            cand = lo + (1 << (30 - j))
            cnt = jnp.sum((bits >= cand).astype(jnp.int32))
            return jnp.where(cnt >= k, cand, lo)

        thr = jax.lax.fori_loop(0, 31, body, jnp.int32(0))
        tval = jax.lax.bitcast_convert_type(thr, jnp.float32)
        gt = bits > thr
        cnt_gt = jnp.sum(gt.astype(jnp.int32))
        sum_gt = jnp.sum(jnp.where(gt, vals, 0.0))
        out_ref[0, 0] = (sum_gt + (k - cnt_gt).astype(jnp.float32) * tval) / k


@functools.partial(jax.jit, static_argnames=("interpret",))
def kernel(input, target, interpret=False):
    n, c = input.shape
    nblocks = n // _R
    k = max(1, int(n * _RATIO))
    out = pl.pallas_call(
        functools.partial(_ohem_body, nblocks=nblocks, k=k),
        grid=(nblocks,),
        in_specs=[
            pl.BlockSpec((_R, c), lambda i: (i, 0)),
            pl.BlockSpec((1, 1, _R), lambda i: (i, 0, 0)),
        ],
        out_specs=pl.BlockSpec(memory_space=pltpu.SMEM),
        out_shape=jax.ShapeDtypeStruct((1, 1), jnp.float32),
        scratch_shapes=[pltpu.VMEM((nblocks, _R), jnp.float32)],
        compiler_params=pltpu.CompilerParams(
            dimension_semantics=("arbitrary",),
        ),
        interpret=interpret,
    )(input, target.reshape(nblocks, 1, _R))
    return out[0, 0]


# speedup vs baseline: 1.3230x; 1.0167x over previous
import functools

import jax
import jax.numpy as jnp
from jax.experimental import pallas as pl
from jax.experimental.pallas import tpu as pltpu

_RATIO = 0.05
_R = 1024


def _ohem_body(x_ref, t_ref, out_ref, loss_ref, *, nblocks, k):
    i = pl.program_id(0)
    r, c = x_ref.shape
    nq = c // 128
    tgt = t_ref[0, 0, :]
    lane = jax.lax.broadcasted_iota(jnp.int32, (r, 128), 1)
    lmask = lane == (tgt[:, None] & 127)
    tq = tgt[:, None] >> 7

    def tree(items, combine):
        while len(items) > 1:
            nxt = [combine(items[p], items[p + 1]) for p in range(0, len(items) - 1, 2)]
            if len(items) % 2:
                nxt.append(items[-1])
            items = nxt
        return items[0]

    chunks = [x_ref[:, 128 * q : 128 * (q + 1)] for q in range(nq)]
    m128 = tree(chunks, jnp.maximum)
    m = jnp.max(m128, axis=1, keepdims=True)

    sel = chunks
    level = 0
    while len(sel) > 1:
        bit = ((tq >> level) & 1) == 1
        sel = [jnp.where(bit, sel[p + 1], sel[p]) if p + 1 < len(sel) else sel[p]
               for p in range(0, len(sel), 2)]
        level += 1
    t_logit = jnp.sum(jnp.where(lmask, sel[0], 0.0), axis=1)

    e128 = tree(
        [jnp.exp(x_ref[:, 128 * q : 128 * (q + 1)] - m) for q in range(nq)],
        jnp.add,
    )
    s = jnp.sum(e128, axis=1)
    lse = m[:, 0] + jnp.log(s)
    loss = lse - t_logit
    loss_ref[pl.ds(i, 1), :] = loss.reshape(1, -1)

    @pl.when(i == nblocks - 1)
    def _select():
        vals = loss_ref[...]
        bits = jax.lax.bitcast_convert_type(vals, jnp.int32)

        def body(j, lo):
            cand = lo + (1 << (30 - j))
            cnt = jnp.sum((bits >= cand).astype(jnp.int32))
            return jnp.where(cnt >= k, cand, lo)

        thr = jax.lax.fori_loop(0, 31, body, jnp.int32(0))
        tval = jax.lax.bitcast_convert_type(thr, jnp.float32)
        gt = bits > thr
        cnt_gt = jnp.sum(gt.astype(jnp.int32))
        sum_gt = jnp.sum(jnp.where(gt, vals, 0.0))
        out_ref[0, 0] = (sum_gt + (k - cnt_gt).astype(jnp.float32) * tval) / k


@functools.partial(jax.jit, static_argnames=("interpret",))
def kernel(input, target, interpret=False):
    n, c = input.shape
    nblocks = n // _R
    k = max(1, int(n * _RATIO))
    out = pl.pallas_call(
        functools.partial(_ohem_body, nblocks=nblocks, k=k),
        grid=(nblocks,),
        in_specs=[
            pl.BlockSpec((_R, c), lambda i: (i, 0)),
            pl.BlockSpec((1, 1, _R), lambda i: (i, 0, 0)),
        ],
        out_specs=pl.BlockSpec(memory_space=pltpu.SMEM),
        out_shape=jax.ShapeDtypeStruct((1, 1), jnp.float32),
        scratch_shapes=[pltpu.VMEM((nblocks, _R), jnp.float32)],
        compiler_params=pltpu.CompilerParams(
            dimension_semantics=("arbitrary",),
        ),
        interpret=interpret,
    )(input, target.reshape(nblocks, 1, _R))
    return out[0, 0]
